# Initial kernel scaffold; baseline (speedup 1.0000x reference)
import jax
import jax.numpy as jnp
from jax.experimental import pallas as pl


def kernel(x, W):
    raise NotImplementedError("write your pallas kernel here")


# speedup vs baseline: 6.3893x; 6.3893x over previous
import functools

import jax
import jax.numpy as jnp
import numpy as np
from jax.experimental import pallas as pl
from jax.experimental.pallas import tpu as pltpu

K = 20
NEG_SLOPE = 0.2


def _edgeconv_block(xb_ref, rows_ref, w1t_ref, w21t_ref, out_ref, *, k, jc):
    xb = xb_ref[0]
    rows = rows_ref[0]
    r = rows.shape[0]
    n = xb.shape[0]

    g = jax.lax.dot_general(rows, xb, (((1,), (1,)), ((), ())),
                            preferred_element_type=jnp.float32)
    xx_all = jnp.sum(xb * xb, axis=1)
    xx_rows = jnp.sum(rows * rows, axis=1)
    nd = 2.0 * g - xx_rows[:, None] - xx_all[None, :]

    y = jnp.dot(xb, w1t_ref[...], preferred_element_type=jnp.float32)
    z = jnp.dot(rows, w21t_ref[...], preferred_element_type=jnp.float32)

    o = y.shape[1]
    neg = jnp.float32(-jnp.inf)

    iota = jax.lax.broadcasted_iota(jnp.int32, (r, n), 1)

    def tk_body(_, carry):
        work, m = carry
        cur = jnp.max(work, axis=1)
        eq = work == cur[:, None]
        jmin = jnp.min(jnp.where(eq, iota, n), axis=1)
        onehot = iota == jmin[:, None]
        sel = jnp.dot(onehot.astype(jnp.float32), y,
                      preferred_element_type=jnp.float32)
        return jnp.where(onehot, neg, work), jnp.maximum(m, sel)

    _, m = jax.lax.fori_loop(
        0, k, tk_body, (nd, jnp.full((r, o), neg, jnp.float32)))
    res = m + z
    out_ref[0] = jnp.maximum(res, NEG_SLOPE * res)


def _edgeconv_tc(x_t, w1t, w21t, *, rblk, jc):
    b, n, c = x_t.shape
    o = w1t.shape[1]
    grid = (b, n // rblk)
    return pl.pallas_call(
        functools.partial(_edgeconv_block, k=K, jc=jc),
        grid=grid,
        in_specs=[
            pl.BlockSpec((1, n, c), lambda bi, ri: (bi, 0, 0)),
            pl.BlockSpec((1, rblk, c), lambda bi, ri: (bi, ri, 0)),
            pl.BlockSpec((c, o), lambda bi, ri: (0, 0)),
            pl.BlockSpec((c, o), lambda bi, ri: (0, 0)),
        ],
        out_specs=pl.BlockSpec((1, rblk, o), lambda bi, ri: (bi, ri, 0)),
        out_shape=jax.ShapeDtypeStruct((b, n, o), jnp.float32),
    )(x_t, x_t, w1t, w21t)


@jax.jit
def kernel(x, W):
    b, c, n = x.shape
    o = W.shape[0]
    x_t = jnp.transpose(x, (0, 2, 1))
    w1 = W[:, :c]
    w2 = W[:, c:]
    w1t = jnp.transpose(w1)
    w21t = jnp.transpose(w2 - w1)
    out = _edgeconv_tc(x_t, w1t, w21t, rblk=256, jc=64)
    return jnp.transpose(out, (0, 2, 1))
